# Initial kernel scaffold; baseline (speedup 1.0000x reference)
import functools

import jax
import jax.numpy as jnp
from jax import lax
from jax.experimental import pallas as pl
from jax.experimental.pallas import tpu as pltpu
from jax.experimental.pallas import tpu_sc as plsc

B = 4096
NG = 20
EMBED = 64
L = 16
NC = 2
NS = 16
NW = NC * NS
BPW = B // NW
CH = 64
NCHUNK = BPW // CH
JC = CH * NG
IDX_BLK = 128
NBLK = JC // IDX_BLK


def _sc_body(x_hbm, y_hbm, v_hbm, w_hbm, out_hbm,
             x_idx, y_idx, vrows, wrows, out_v, sem):
    cid = lax.axis_index("c")
    sid = lax.axis_index("s")
    wid = sid * NC + cid

    pltpu.sync_copy(x_hbm.at[pl.ds(wid * BPW, BPW)], x_idx)
    pltpu.async_copy(v_hbm.at[x_idx], vrows, sem).wait()

    def chunk_body(c, carry):
        base = wid * (BPW * NG) + c * JC
        pltpu.sync_copy(y_hbm.at[pl.ds(base, JC)], y_idx)
        copies = [
            pltpu.async_copy(
                w_hbm.at[y_idx.at[pl.ds(j * IDX_BLK, IDX_BLK)]],
                wrows.at[pl.ds(j * IDX_BLK, IDX_BLK)],
                sem,
            )
            for j in range(NBLK)
        ]
        for cp in copies:
            cp.wait()

        def bag_body(i, carry2):
            bi = c * CH + i
            v0 = vrows[bi, pl.ds(0, L)]
            v1 = vrows[bi, pl.ds(L, L)]
            v2 = vrows[bi, pl.ds(2 * L, L)]
            v3 = vrows[bi, pl.ds(3 * L, L)]
            for g in range(NG):
                r = i * NG + g
                acc = (v0 * wrows[r, pl.ds(0, L)]
                       + v1 * wrows[r, pl.ds(L, L)]
                       + v2 * wrows[r, pl.ds(2 * L, L)]
                       + v3 * wrows[r, pl.ds(3 * L, L)])
                out_v[r] = jnp.sum(acc)
            return carry2

        lax.fori_loop(0, CH, bag_body, 0, unroll=False)
        pltpu.sync_copy(out_v, out_hbm.at[pl.ds(base, JC)])
        return carry

    lax.fori_loop(0, NCHUNK, chunk_body, 0, unroll=False)


@jax.jit
def _run(x, y, v_table, w_table):
    mesh = plsc.VectorSubcoreMesh(core_axis_name="c", subcore_axis_name="s")
    out = pl.kernel(
        _sc_body,
        out_type=jax.ShapeDtypeStruct((B * NG,), jnp.float32),
        mesh=mesh,
        scratch_types=[
            pltpu.VMEM((BPW,), jnp.int32),
            pltpu.VMEM((JC,), jnp.int32),
            pltpu.VMEM((BPW, EMBED), jnp.float32),
            pltpu.VMEM((JC, EMBED), jnp.float32),
            pltpu.VMEM((JC,), jnp.float32),
            pltpu.SemaphoreType.DMA,
        ],
    )(x, y, v_table, w_table)
    return out.reshape(B, NG)


def kernel(x, y, xoffsets, yoffsets, V_table, W_table):
    return _run(x.astype(jnp.int32), y.astype(jnp.int32), V_table, W_table)


# speedup vs baseline: 3.7700x; 3.7700x over previous
import functools

import jax
import jax.numpy as jnp
from jax import lax
from jax.experimental import pallas as pl
from jax.experimental.pallas import tpu as pltpu
from jax.experimental.pallas import tpu_sc as plsc

B = 4096
NG = 20
EMBED = 64
L = 16
NC = 2
NS = 16
NW = NC * NS
BPW = B // NW
CH = 64
NCHUNK = BPW // CH
JC = CH * NG
IDX_BLK = 128
NBLK = JC // IDX_BLK


def _sc_body(x_hbm, y_hbm, v_hbm, w_hbm, out_hbm,
             x_idx, y_idx, vrows, wrows, out_v, sem):
    cid = lax.axis_index("c")
    sid = lax.axis_index("s")
    wid = sid * NC + cid

    pltpu.sync_copy(x_hbm.at[pl.ds(wid * BPW, BPW)], x_idx)
    pltpu.async_copy(v_hbm.at[x_idx], vrows, sem).wait()

    def chunk_body(c, carry):
        base = wid * (BPW * NG) + c * JC
        pltpu.sync_copy(y_hbm.at[pl.ds(base, JC)], y_idx)
        copies = [
            pltpu.async_copy(
                w_hbm.at[y_idx.at[pl.ds(j * IDX_BLK, IDX_BLK)]],
                wrows.at[pl.ds(j * IDX_BLK, IDX_BLK)],
                sem,
            )
            for j in range(NBLK)
        ]
        for cp in copies:
            cp.wait()

        row_ids = lax.iota(jnp.int32, L)
        lane_masks = [row_ids == l for l in range(L)]
        perms = [row_ids ^ k for k in (8, 4, 2, 1)]

        def quad_body(q, carry2):
            vbase = c * CH + q * 4
            vv = [[vrows[vbase + b, pl.ds(k * L, L)] for k in range(4)]
                  for b in range(4)]
            rbase = q * 80
            for m in range(5):
                tot = jnp.zeros((L,), jnp.float32)
                for l in range(L):
                    t = m * L + l
                    b = t // NG
                    r = rbase + t
                    acc = (vv[b][0] * wrows[r, pl.ds(0, L)]
                           + vv[b][1] * wrows[r, pl.ds(L, L)]
                           + vv[b][2] * wrows[r, pl.ds(2 * L, L)]
                           + vv[b][3] * wrows[r, pl.ds(3 * L, L)])
                    for p in perms:
                        acc = acc + acc[p]
                    tot = jnp.where(lane_masks[l], acc, tot)
                out_v[pl.ds(rbase + m * L, L)] = tot
            return carry2

        lax.fori_loop(0, CH // 4, quad_body, 0, unroll=False)
        pltpu.sync_copy(out_v, out_hbm.at[pl.ds(base, JC)])
        return carry

    lax.fori_loop(0, NCHUNK, chunk_body, 0, unroll=False)


@jax.jit
def _run(x, y, v_table, w_table):
    mesh = plsc.VectorSubcoreMesh(core_axis_name="c", subcore_axis_name="s")
    out = pl.kernel(
        _sc_body,
        out_type=jax.ShapeDtypeStruct((B * NG,), jnp.float32),
        mesh=mesh,
        scratch_types=[
            pltpu.VMEM((BPW,), jnp.int32),
            pltpu.VMEM((JC,), jnp.int32),
            pltpu.VMEM((BPW, EMBED), jnp.float32),
            pltpu.VMEM((JC, EMBED), jnp.float32),
            pltpu.VMEM((JC,), jnp.float32),
            pltpu.SemaphoreType.DMA,
        ],
        compiler_params=pltpu.CompilerParams(use_tc_tiling_on_sc=False),
    )(x, y, v_table, w_table)
    return out.reshape(B, NG)


def kernel(x, y, xoffsets, yoffsets, V_table, W_table):
    return _run(x.astype(jnp.int32), y.astype(jnp.int32), V_table, W_table)


# speedup vs baseline: 5.6755x; 1.5054x over previous
import functools

import jax
import jax.numpy as jnp
from jax import lax
from jax.experimental import pallas as pl
from jax.experimental.pallas import tpu as pltpu
from jax.experimental.pallas import tpu_sc as plsc

B = 4096
NG = 20
EMBED = 64
L = 16
NC = 2
NS = 16
NW = NC * NS
BPW = B // NW
CH = 64
NCHUNK = BPW // CH
JC = CH * NG
IDX_BLK = 128
NBLK = JC // IDX_BLK

TCK = 64
BPS = 512
NSTEP = B // BPS
ROUNDS = BPS // TCK


def _tc_gather_body(x_ref, vt_ref, out_ref, ring, sem):
    g = pl.program_id(0)
    base = g * BPS
    col_iota = lax.broadcasted_iota(jnp.int32, (EMBED, 128), 1)

    def start(i, k):
        blk = x_ref[base + i] // 128
        pltpu.make_async_copy(
            vt_ref.at[:, pl.ds(blk * 128, 128)], ring.at[k], sem.at[k]
        ).start()

    for k in range(TCK):
        start(k, k)

    def round_body(r, carry):
        for k in range(TCK):
            i = r * TCK + k
            pltpu.make_async_copy(
                vt_ref.at[:, pl.ds(0, 128)], ring.at[k], sem.at[k]
            ).wait()
            co = lax.rem(x_ref[base + i], 128)
            bval = ring[k]
            row = jnp.where(col_iota == co, bval, 0.0).sum(axis=1)
            out_ref[i, :] = row

            @pl.when(r + 1 < ROUNDS)
            def _():
                start((r + 1) * TCK + k, k)

        return carry

    lax.fori_loop(0, ROUNDS, round_body, 0, unroll=False)


def _sc_body(y_hbm, phi_hbm, w_hbm, out_hbm,
             y_idx, vrows, wrows, out_v, sem):
    cid = lax.axis_index("c")
    sid = lax.axis_index("s")
    wid = sid * NC + cid

    pltpu.sync_copy(phi_hbm.at[pl.ds(wid * BPW, BPW), :], vrows)

    def chunk_body(c, carry):
        base = wid * (BPW * NG) + c * JC
        pltpu.sync_copy(y_hbm.at[pl.ds(base, JC)], y_idx)
        copies = [
            pltpu.async_copy(
                w_hbm.at[y_idx.at[pl.ds(j * IDX_BLK, IDX_BLK)]],
                wrows.at[pl.ds(j * IDX_BLK, IDX_BLK)],
                sem,
            )
            for j in range(NBLK)
        ]
        for cp in copies:
            cp.wait()

        row_ids = lax.iota(jnp.int32, L)
        lane_masks = [row_ids == l for l in range(L)]
        perms = [row_ids ^ k for k in (8, 4, 2, 1)]

        def quad_body(q, carry2):
            vbase = c * CH + q * 4
            vv = [[vrows[vbase + b, pl.ds(k * L, L)] for k in range(4)]
                  for b in range(4)]
            rbase = q * 80
            for m in range(5):
                tot = jnp.zeros((L,), jnp.float32)
                for l in range(L):
                    t = m * L + l
                    b = t // NG
                    r = rbase + t
                    acc = (vv[b][0] * wrows[r, pl.ds(0, L)]
                           + vv[b][1] * wrows[r, pl.ds(L, L)]
                           + vv[b][2] * wrows[r, pl.ds(2 * L, L)]
                           + vv[b][3] * wrows[r, pl.ds(3 * L, L)])
                    for p in perms:
                        acc = acc + acc[p]
                    tot = jnp.where(lane_masks[l], acc, tot)
                out_v[pl.ds(rbase + m * L, L)] = tot
            return carry2

        lax.fori_loop(0, CH // 4, quad_body, 0, unroll=False)
        pltpu.sync_copy(out_v, out_hbm.at[pl.ds(base, JC)])
        return carry

    lax.fori_loop(0, NCHUNK, chunk_body, 0, unroll=False)


@jax.jit
def _run(x, y, v_table, w_table):
    phi_x = pl.pallas_call(
        _tc_gather_body,
        grid=(NSTEP,),
        in_specs=[
            pl.BlockSpec(memory_space=pltpu.SMEM),
            pl.BlockSpec(memory_space=pl.ANY),
        ],
        out_specs=pl.BlockSpec((BPS, EMBED), lambda g: (g, 0)),
        out_shape=jax.ShapeDtypeStruct((B, EMBED), jnp.float32),
        scratch_shapes=[
            pltpu.VMEM((TCK, EMBED, 128), jnp.float32),
            pltpu.SemaphoreType.DMA((TCK,)),
        ],
    )(x, v_table.T)

    mesh = plsc.VectorSubcoreMesh(core_axis_name="c", subcore_axis_name="s")
    out = pl.kernel(
        _sc_body,
        out_type=jax.ShapeDtypeStruct((B * NG,), jnp.float32),
        mesh=mesh,
        scratch_types=[
            pltpu.VMEM((JC,), jnp.int32),
            pltpu.VMEM((BPW, EMBED), jnp.float32),
            pltpu.VMEM((JC, EMBED), jnp.float32),
            pltpu.VMEM((JC,), jnp.float32),
            pltpu.SemaphoreType.DMA,
        ],
        compiler_params=pltpu.CompilerParams(use_tc_tiling_on_sc=False),
    )(y, phi_x, w_table)
    return out.reshape(B, NG)


def kernel(x, y, xoffsets, yoffsets, V_table, W_table):
    return _run(x.astype(jnp.int32), y.astype(jnp.int32), V_table, W_table)


# speedup vs baseline: 5.9355x; 1.0458x over previous
import functools

import jax
import jax.numpy as jnp
from jax import lax
from jax.experimental import pallas as pl
from jax.experimental.pallas import tpu as pltpu
from jax.experimental.pallas import tpu_sc as plsc

B = 4096
NG = 20
EMBED = 64
L = 16
NC = 2
NS = 16
NW = NC * NS
BPW = B // NW
CH = 64
NCHUNK = BPW // CH
JC = CH * NG
IDX_BLK = 128
NBLK = JC // IDX_BLK

TCK = 64
BPS = 128
NSTEP = B // BPS
ROUNDS = BPS // TCK


def _tc_gather_body(x_ref, vt_ref, out_ref, ring, sem):
    g = pl.program_id(0)
    base = g * BPS
    lane_iota = lax.broadcasted_iota(jnp.int32, (1, BPS), 1)

    def start(i, k):
        blk = x_ref[base + i] // 128
        pltpu.make_async_copy(
            vt_ref.at[:, pl.ds(blk * 128, 128)], ring.at[k], sem.at[k]
        ).start()

    for k in range(TCK):
        start(k, k)

    acc = jnp.zeros((EMBED, BPS), jnp.float32)
    for r in range(ROUNDS):
        for k in range(TCK):
            i = r * TCK + k
            pltpu.make_async_copy(
                vt_ref.at[:, pl.ds(0, 128)], ring.at[k], sem.at[k]
            ).wait()
            co = lax.rem(x_ref[base + i], 128)
            rolled = pltpu.roll(ring[k], i - co, axis=1)
            acc = jnp.where(lane_iota == i, rolled, acc)
            if r + 1 < ROUNDS:
                start((r + 1) * TCK + k, k)
    out_ref[:, :] = acc


def _sc_body(y_hbm, phi_hbm, w_hbm, out_hbm,
             y_idx, vrows, wrows, out_v, sem):
    cid = lax.axis_index("c")
    sid = lax.axis_index("s")
    wid = sid * NC + cid

    pltpu.sync_copy(phi_hbm.at[pl.ds(wid * BPW, BPW), :], vrows)

    def chunk_body(c, carry):
        base = wid * (BPW * NG) + c * JC
        pltpu.sync_copy(y_hbm.at[pl.ds(base, JC)], y_idx)
        copies = [
            pltpu.async_copy(
                w_hbm.at[y_idx.at[pl.ds(j * IDX_BLK, IDX_BLK)]],
                wrows.at[pl.ds(j * IDX_BLK, IDX_BLK)],
                sem,
            )
            for j in range(NBLK)
        ]
        for cp in copies:
            cp.wait()

        row_ids = lax.iota(jnp.int32, L)
        lane_masks = [row_ids == l for l in range(L)]
        perms = [row_ids ^ k for k in (8, 4, 2, 1)]

        def quad_body(q, carry2):
            vbase = c * CH + q * 4
            vv = [[vrows[vbase + b, pl.ds(k * L, L)] for k in range(4)]
                  for b in range(4)]
            rbase = q * 80
            for m in range(5):
                tot = jnp.zeros((L,), jnp.float32)
                for l in range(L):
                    t = m * L + l
                    b = t // NG
                    r = rbase + t
                    acc = (vv[b][0] * wrows[r, pl.ds(0, L)]
                           + vv[b][1] * wrows[r, pl.ds(L, L)]
                           + vv[b][2] * wrows[r, pl.ds(2 * L, L)]
                           + vv[b][3] * wrows[r, pl.ds(3 * L, L)])
                    for p in perms:
                        acc = acc + acc[p]
                    tot = jnp.where(lane_masks[l], acc, tot)
                out_v[pl.ds(rbase + m * L, L)] = tot
            return carry2

        lax.fori_loop(0, CH // 4, quad_body, 0, unroll=False)
        pltpu.sync_copy(out_v, out_hbm.at[pl.ds(base, JC)])
        return carry

    lax.fori_loop(0, NCHUNK, chunk_body, 0, unroll=False)


@jax.jit
def _run(x, y, v_table, w_table):
    phi_xt = pl.pallas_call(
        _tc_gather_body,
        grid=(NSTEP,),
        in_specs=[
            pl.BlockSpec(memory_space=pltpu.SMEM),
            pl.BlockSpec(memory_space=pl.ANY),
        ],
        out_specs=pl.BlockSpec((EMBED, BPS), lambda g: (0, g)),
        out_shape=jax.ShapeDtypeStruct((EMBED, B), jnp.float32),
        scratch_shapes=[
            pltpu.VMEM((TCK, EMBED, 128), jnp.float32),
            pltpu.SemaphoreType.DMA((TCK,)),
        ],
    )(x, v_table.T)
    phi_x = phi_xt.T

    mesh = plsc.VectorSubcoreMesh(core_axis_name="c", subcore_axis_name="s")
    out = pl.kernel(
        _sc_body,
        out_type=jax.ShapeDtypeStruct((B * NG,), jnp.float32),
        mesh=mesh,
        scratch_types=[
            pltpu.VMEM((JC,), jnp.int32),
            pltpu.VMEM((BPW, EMBED), jnp.float32),
            pltpu.VMEM((JC, EMBED), jnp.float32),
            pltpu.VMEM((JC,), jnp.float32),
            pltpu.SemaphoreType.DMA,
        ],
        compiler_params=pltpu.CompilerParams(use_tc_tiling_on_sc=False),
    )(y, phi_x, w_table)
    return out.reshape(B, NG)


def kernel(x, y, xoffsets, yoffsets, V_table, W_table):
    return _run(x.astype(jnp.int32), y.astype(jnp.int32), V_table, W_table)


# speedup vs baseline: 5.9403x; 1.0008x over previous
import functools

import jax
import jax.numpy as jnp
from jax import lax
from jax.experimental import pallas as pl
from jax.experimental.pallas import tpu as pltpu
from jax.experimental.pallas import tpu_sc as plsc

B = 4096
NG = 20
EMBED = 64
L = 16
NC = 2
NS = 16
NW = NC * NS
BPW = B // NW
CH = 64
NCHUNK = BPW // CH
JC = CH * NG
IDX_BLK = 128
NBLK = JC // IDX_BLK

TCK = 128
BPS = 128
NSTEP = B // BPS
ROUNDS = BPS // TCK


def _tc_gather_body(x_ref, vt_ref, out_ref, ring, sem):
    g = pl.program_id(0)
    base = g * BPS
    lane_iota = lax.broadcasted_iota(jnp.int32, (1, BPS), 1)

    def start(i, k):
        blk = x_ref[base + i] // 128
        pltpu.make_async_copy(
            vt_ref.at[:, pl.ds(blk * 128, 128)], ring.at[k], sem.at[k]
        ).start()

    for k in range(TCK):
        start(k, k)

    acc = jnp.zeros((EMBED, BPS), jnp.float32)
    for r in range(ROUNDS):
        for k in range(TCK):
            i = r * TCK + k
            pltpu.make_async_copy(
                vt_ref.at[:, pl.ds(0, 128)], ring.at[k], sem.at[k]
            ).wait()
            co = lax.rem(x_ref[base + i], 128)
            rolled = pltpu.roll(ring[k], i - co, axis=1)
            acc = jnp.where(lane_iota == i, rolled, acc)
            if r + 1 < ROUNDS:
                start((r + 1) * TCK + k, k)
    out_ref[:, :] = acc


def _sc_body(y_hbm, phi_hbm, w_hbm, out_hbm,
             y_idx, vrows, wrows, out_v, sem):
    cid = lax.axis_index("c")
    sid = lax.axis_index("s")
    wid = sid * NC + cid

    pltpu.sync_copy(phi_hbm.at[pl.ds(wid * BPW, BPW), :], vrows)

    def chunk_body(c, carry):
        base = wid * (BPW * NG) + c * JC
        pltpu.sync_copy(y_hbm.at[pl.ds(base, JC)], y_idx)
        copies = [
            pltpu.async_copy(
                w_hbm.at[y_idx.at[pl.ds(j * IDX_BLK, IDX_BLK)]],
                wrows.at[pl.ds(j * IDX_BLK, IDX_BLK)],
                sem,
            )
            for j in range(NBLK)
        ]
        for cp in copies:
            cp.wait()

        row_ids = lax.iota(jnp.int32, L)
        lane_masks = [row_ids == l for l in range(L)]
        perms = [row_ids ^ k for k in (8, 4, 2, 1)]

        def quad_body(q, carry2):
            vbase = c * CH + q * 4
            vv = [[vrows[vbase + b, pl.ds(k * L, L)] for k in range(4)]
                  for b in range(4)]
            rbase = q * 80
            for m in range(5):
                tot = jnp.zeros((L,), jnp.float32)
                for l in range(L):
                    t = m * L + l
                    b = t // NG
                    r = rbase + t
                    acc = (vv[b][0] * wrows[r, pl.ds(0, L)]
                           + vv[b][1] * wrows[r, pl.ds(L, L)]
                           + vv[b][2] * wrows[r, pl.ds(2 * L, L)]
                           + vv[b][3] * wrows[r, pl.ds(3 * L, L)])
                    for p in perms:
                        acc = acc + acc[p]
                    tot = jnp.where(lane_masks[l], acc, tot)
                out_v[pl.ds(rbase + m * L, L)] = tot
            return carry2

        lax.fori_loop(0, CH // 4, quad_body, 0, unroll=False)
        pltpu.sync_copy(out_v, out_hbm.at[pl.ds(base, JC)])
        return carry

    lax.fori_loop(0, NCHUNK, chunk_body, 0, unroll=False)


@jax.jit
def _run(x, y, v_table, w_table):
    phi_xt = pl.pallas_call(
        _tc_gather_body,
        grid=(NSTEP,),
        in_specs=[
            pl.BlockSpec(memory_space=pltpu.SMEM),
            pl.BlockSpec(memory_space=pl.ANY),
        ],
        out_specs=pl.BlockSpec((EMBED, BPS), lambda g: (0, g)),
        out_shape=jax.ShapeDtypeStruct((EMBED, B), jnp.float32),
        scratch_shapes=[
            pltpu.VMEM((TCK, EMBED, 128), jnp.float32),
            pltpu.SemaphoreType.DMA((TCK,)),
        ],
    )(x, v_table.T)
    phi_x = phi_xt.T

    mesh = plsc.VectorSubcoreMesh(core_axis_name="c", subcore_axis_name="s")
    out = pl.kernel(
        _sc_body,
        out_type=jax.ShapeDtypeStruct((B * NG,), jnp.float32),
        mesh=mesh,
        scratch_types=[
            pltpu.VMEM((JC,), jnp.int32),
            pltpu.VMEM((BPW, EMBED), jnp.float32),
            pltpu.VMEM((JC, EMBED), jnp.float32),
            pltpu.VMEM((JC,), jnp.float32),
            pltpu.SemaphoreType.DMA,
        ],
        compiler_params=pltpu.CompilerParams(use_tc_tiling_on_sc=False),
    )(y, phi_x, w_table)
    return out.reshape(B, NG)


def kernel(x, y, xoffsets, yoffsets, V_table, W_table):
    return _run(x.astype(jnp.int32), y.astype(jnp.int32), V_table, W_table)


# speedup vs baseline: 5.9674x; 1.0046x over previous
import functools

import jax
import jax.numpy as jnp
from jax import lax
from jax.experimental import pallas as pl
from jax.experimental.pallas import tpu as pltpu
from jax.experimental.pallas import tpu_sc as plsc

B = 4096
NG = 20
EMBED = 64
L = 16
NC = 2
NS = 16
NW = NC * NS
BPW = B // NW
CH = 64
NCHUNK = BPW // CH
JC = CH * NG
IDX_BLK = 128
NBLK = JC // IDX_BLK

BPS = 128
NSTEP = B // BPS


def _tc_gather_body(x_ref, vt_ref, out_ref, ring, sems):
    g = pl.program_id(0)
    lane_iota = lax.broadcasted_iota(jnp.int32, (1, BPS), 1)

    def issue(step, par):
        base = step * BPS
        for k in range(BPS):
            blk = x_ref[base + k] // 128
            pltpu.make_async_copy(
                vt_ref.at[:, pl.ds(blk * 128, 128)],
                ring.at[par, k],
                sems.at[par, k],
            ).start()

    @pl.when(g == 0)
    def _():
        issue(0, 0)

    @pl.when(g + 1 < NSTEP)
    def _():
        issue(g + 1, lax.rem(g + 1, 2))

    par = lax.rem(g, 2)
    base = g * BPS
    acc = jnp.zeros((EMBED, BPS), jnp.float32)
    for k in range(BPS):
        pltpu.make_async_copy(
            vt_ref.at[:, pl.ds(0, 128)], ring.at[par, k], sems.at[par, k]
        ).wait()
        co = lax.rem(x_ref[base + k], 128)
        rolled = pltpu.roll(ring[par, k], k - co, axis=1)
        acc = jnp.where(lane_iota == k, rolled, acc)
    out_ref[:, :] = acc


def _sc_body(y_hbm, phi_hbm, w_hbm, out_hbm,
             y_idx, vrows, wrows, out_v, sem):
    cid = lax.axis_index("c")
    sid = lax.axis_index("s")
    wid = sid * NC + cid

    pltpu.sync_copy(phi_hbm.at[pl.ds(wid * BPW, BPW), :], vrows)

    def chunk_body(c, carry):
        base = wid * (BPW * NG) + c * JC
        pltpu.sync_copy(y_hbm.at[pl.ds(base, JC)], y_idx)
        copies = [
            pltpu.async_copy(
                w_hbm.at[y_idx.at[pl.ds(j * IDX_BLK, IDX_BLK)]],
                wrows.at[pl.ds(j * IDX_BLK, IDX_BLK)],
                sem,
            )
            for j in range(NBLK)
        ]
        for cp in copies:
            cp.wait()

        row_ids = lax.iota(jnp.int32, L)
        lane_masks = [row_ids == l for l in range(L)]
        perms = [row_ids ^ k for k in (8, 4, 2, 1)]

        def quad_body(q, carry2):
            vbase = c * CH + q * 4
            vv = [[vrows[vbase + b, pl.ds(k * L, L)] for k in range(4)]
                  for b in range(4)]
            rbase = q * 80
            for m in range(5):
                tot = jnp.zeros((L,), jnp.float32)
                for l in range(L):
                    t = m * L + l
                    b = t // NG
                    r = rbase + t
                    acc = (vv[b][0] * wrows[r, pl.ds(0, L)]
                           + vv[b][1] * wrows[r, pl.ds(L, L)]
                           + vv[b][2] * wrows[r, pl.ds(2 * L, L)]
                           + vv[b][3] * wrows[r, pl.ds(3 * L, L)])
                    for p in perms:
                        acc = acc + acc[p]
                    tot = jnp.where(lane_masks[l], acc, tot)
                out_v[pl.ds(rbase + m * L, L)] = tot
            return carry2

        lax.fori_loop(0, CH // 4, quad_body, 0, unroll=False)
        pltpu.sync_copy(out_v, out_hbm.at[pl.ds(base, JC)])
        return carry

    lax.fori_loop(0, NCHUNK, chunk_body, 0, unroll=False)


@jax.jit
def _run(x, y, v_table, w_table):
    phi_xt = pl.pallas_call(
        _tc_gather_body,
        grid=(NSTEP,),
        in_specs=[
            pl.BlockSpec(memory_space=pltpu.SMEM),
            pl.BlockSpec(memory_space=pl.ANY),
        ],
        out_specs=pl.BlockSpec((EMBED, BPS), lambda g: (0, g)),
        out_shape=jax.ShapeDtypeStruct((EMBED, B), jnp.float32),
        scratch_shapes=[
            pltpu.VMEM((2, BPS, EMBED, 128), jnp.float32),
            pltpu.SemaphoreType.DMA((2, BPS)),
        ],
    )(x, v_table.T)
    phi_x = phi_xt.T

    mesh = plsc.VectorSubcoreMesh(core_axis_name="c", subcore_axis_name="s")
    out = pl.kernel(
        _sc_body,
        out_type=jax.ShapeDtypeStruct((B * NG,), jnp.float32),
        mesh=mesh,
        scratch_types=[
            pltpu.VMEM((JC,), jnp.int32),
            pltpu.VMEM((BPW, EMBED), jnp.float32),
            pltpu.VMEM((JC, EMBED), jnp.float32),
            pltpu.VMEM((JC,), jnp.float32),
            pltpu.SemaphoreType.DMA,
        ],
        compiler_params=pltpu.CompilerParams(use_tc_tiling_on_sc=False),
    )(y, phi_x, w_table)
    return out.reshape(B, NG)


def kernel(x, y, xoffsets, yoffsets, V_table, W_table):
    return _run(x.astype(jnp.int32), y.astype(jnp.int32), V_table, W_table)
